# baseline (speedup 1.0000x reference)
import functools
import math

import jax
import jax.numpy as jnp
from jax import lax
from jax.experimental import pallas as pl
from jax.experimental.pallas import tpu as pltpu

_EPS = 0.05


def _auc_kernel(predsT_ref, trow_ref, out_ref):
    C, B = predsT_ref.shape
    c2 = math.exp(_EPS)
    c1 = 1.0 + math.exp(2 * _EPS)

    zt = predsT_ref[...]
    et = jnp.exp(zt - jnp.max(zt, axis=0, keepdims=True))
    pt = et / jnp.sum(et, axis=0, keepdims=True)
    clsr = lax.broadcasted_iota(jnp.int32, (C, B), 0)
    same = trow_ref[...] == clsr
    ohT = same.astype(jnp.float32)
    counts = jnp.sum(ohT, axis=1, keepdims=True)
    denom = counts * (float(B) - counts)
    facc = jnp.where(denom > 0.0, 1.0 / denom, 0.0)
    fac = jnp.sum(ohT * facc, axis=0, keepdims=True)
    q = jnp.sum(ohT * pt, axis=0, keepdims=True)
    a = c2 * jnp.exp(4.0 - 4.0 * q)
    b = c2 * jnp.exp(4.0 * q - 4.0)
    e4 = jnp.where(same, 0.0, jnp.exp(4.0 * pt))
    e4m = jnp.where(same, 0.0, jnp.exp(-4.0 * pt))
    ones = jnp.ones((1, B), jnp.float32)
    zeros = jnp.zeros((1, B), jnp.float32)
    m1 = jnp.concatenate(
        [e4, e4m, c1 * ones, ohT, zeros], axis=0)
    m2 = jnp.concatenate(
        [a * ohT, b * ohT, ones, (1.0 - c1) * ohT, zeros], axis=0)

    arg = lax.dot_general(
        m1, m2,
        dimension_numbers=(((0,), (0,)), ((), ())),
        precision=lax.Precision.DEFAULT)
    R = B // 8
    p0 = arg[0 * R:1 * R] * arg[1 * R:2 * R]
    p1 = arg[2 * R:3 * R] * arg[3 * R:4 * R]
    p2 = arg[4 * R:5 * R] * arg[5 * R:6 * R]
    p3 = arg[6 * R:7 * R] * arg[7 * R:8 * R]
    prod = (p0 * p1) * (p2 * p3)
    colsum = jnp.sum(jnp.log(prod), axis=0, keepdims=True)
    out_ref[...] = jnp.sum(colsum * fac, keepdims=True).reshape(1, 1)


@functools.partial(jax.jit, static_argnames=("interpret",))
def kernel(preds, targets, interpret=False):
    B, C = preds.shape
    preds_t = preds.T
    t_row = targets.astype(jnp.int32).reshape(1, B)
    out = pl.pallas_call(
        _auc_kernel,
        in_specs=[
            pl.BlockSpec((C, B), lambda: (0, 0)),
            pl.BlockSpec((1, B), lambda: (0, 0)),
        ],
        out_specs=pl.BlockSpec((1, 1), lambda: (0, 0)),
        out_shape=jax.ShapeDtypeStruct((1, 1), jnp.float32),
        interpret=interpret,
    )(preds_t, t_row)
    return out.reshape((1,))
